# Initial kernel scaffold; baseline (speedup 1.0000x reference)
import jax
import jax.numpy as jnp
from jax.experimental import pallas as pl


def kernel(x, edge_index, Wl1, Wr1, att1, b1, Wl2, Wr2, att2, b2, Wl3, Wr3, att3, b3):
    raise NotImplementedError("write your pallas kernel here")


# speedup vs baseline: 3.7112x; 3.7112x over previous
import functools

import jax
import jax.numpy as jnp
from jax import lax
from jax.experimental import pallas as pl
from jax.experimental.pallas import tpu as pltpu
from jax.experimental.pallas import tpu_sc as plsc

NN = 10000
CC = 128
EDGES = 160000
EE = EDGES + NN
NWK = 32
WIN = 128
NWIN = 42
PT = WIN * NWIN
EP = NWK * PT
RA = 10112
RT = RA // 16
BN = 400
NB = NN // BN
EB = 1024
NEB = EP // EB
NEG = 0.2
F32 = jnp.float32


def _k1_body(x_ref, wl_ref, wr_ref, xl_ref, xr_ref):
    x = x_ref[...]
    xl_ref[...] = jnp.dot(x, wl_ref[...], preferred_element_type=F32)
    xr_ref[...] = jnp.dot(x, wr_ref[...], preferred_element_type=F32)


@functools.lru_cache(maxsize=None)
def _k1(H):
    return pl.pallas_call(
        _k1_body,
        grid=(H, NB),
        in_specs=[
            pl.BlockSpec((BN, CC), lambda h, nb: (nb, 0)),
            pl.BlockSpec((CC, CC), lambda h, nb: (0, h)),
            pl.BlockSpec((CC, CC), lambda h, nb: (0, h)),
        ],
        out_specs=[
            pl.BlockSpec((BN, CC), lambda h, nb: (h * NB + nb, 0)),
            pl.BlockSpec((BN, CC), lambda h, nb: (h * NB + nb, 0)),
        ],
        out_shape=[
            jax.ShapeDtypeStruct((H * NN, CC), F32),
            jax.ShapeDtypeStruct((H * NN, CC), F32),
        ],
    )


@functools.lru_cache(maxsize=None)
def _k2(H):
    mesh = plsc.VectorSubcoreMesh(core_axis_name="c", subcore_axis_name="s")

    @functools.partial(
        pl.kernel,
        out_type=jax.ShapeDtypeStruct((H * EP, CC), F32),
        mesh=mesh,
        scratch_types=[
            pltpu.VMEM((NWIN, WIN), jnp.int32),
            pltpu.VMEM((NWIN, WIN), jnp.int32),
            pltpu.VMEM((WIN, CC), F32),
            pltpu.SemaphoreType.DMA,
        ],
    )
    def k2(xl_hbm, xr_hbm, srch_hbm, dsth_hbm, g_hbm, sidx, didx, buf, sem):
        cid = lax.axis_index("c")
        sid = lax.axis_index("s")
        wid = cid * 16 + sid

        def head(h, carry):
            pltpu.sync_copy(srch_hbm.at[h * NWK + wid], sidx)
            pltpu.sync_copy(dsth_hbm.at[h * NWK + wid], didx)

            def win(w, carry2):
                pltpu.async_copy(xl_hbm.at[sidx.at[w]], buf, sem).wait()
                pltpu.async_copy(xr_hbm.at[didx.at[w]], buf, sem,
                                 add=True).wait()
                base = h * EP + wid * PT + w * WIN
                pltpu.sync_copy(buf, g_hbm.at[pl.ds(base, WIN)])
                return carry2

            lax.fori_loop(0, NWIN, win, carry)
            return carry

        lax.fori_loop(0, H, head, 0)

    return k2


def _k3a_body(g_ref, att_ref, alpha_ref, gmax_ref):
    eb = pl.program_id(1)
    g = g_ref[...]
    l = jnp.where(g >= 0, g, NEG * g)
    aw = att_ref[pl.program_id(0), :].reshape(1, CC)
    s = (l * aw).sum(axis=1)
    alpha_ref[...] = s

    @pl.when(eb == 0)
    def _():
        gmax_ref[...] = jnp.full((CC,), -jnp.inf, F32)

    gmax_ref[...] = jnp.maximum(gmax_ref[...], jnp.full((CC,), s.max(), F32))


@functools.lru_cache(maxsize=None)
def _k3a(H):
    return pl.pallas_call(
        _k3a_body,
        grid=(H, NEB),
        in_specs=[
            pl.BlockSpec((EB, CC), lambda h, eb: (h * NEB + eb, 0)),
            pl.BlockSpec((H, CC), lambda h, eb: (0, 0)),
        ],
        out_specs=[
            pl.BlockSpec((EB,), lambda h, eb: (h * NEB + eb,)),
            pl.BlockSpec((CC,), lambda h, eb: (h,)),
        ],
        out_shape=[
            jax.ShapeDtypeStruct((H * EP,), F32),
            jax.ShapeDtypeStruct((H * CC,), F32),
        ],
    )


def _k3b_body(alpha_ref, gmax_ref, ea_ref):
    ea_ref[...] = jnp.exp(alpha_ref[...] - gmax_ref[0])


@functools.lru_cache(maxsize=None)
def _k3b(H):
    return pl.pallas_call(
        _k3b_body,
        grid=(H, NEB),
        in_specs=[
            pl.BlockSpec((EB,), lambda h, eb: (h * NEB + eb,)),
            pl.BlockSpec((CC,), lambda h, eb: (h,)),
        ],
        out_specs=pl.BlockSpec((EB,), lambda h, eb: (h * NEB + eb,)),
        out_shape=jax.ShapeDtypeStruct((H * EP,), F32),
    )


@functools.lru_cache(maxsize=None)
def _k4(H):
    mesh = plsc.VectorSubcoreMesh(core_axis_name="c", subcore_axis_name="s")

    @functools.partial(
        pl.kernel,
        out_type=jax.ShapeDtypeStruct((2 * H * RA, CC), F32),
        mesh=mesh,
        scratch_types=[
            pltpu.VMEM((NWIN, WIN), jnp.int32),
            pltpu.VMEM((NWIN, WIN), jnp.int32),
            pltpu.VMEM((NWIN, WIN), F32),
            pltpu.VMEM((WIN, CC), F32),
            pltpu.VMEM_SHARED((RA, CC), F32),
            pltpu.SemaphoreType.DMA,
        ],
    )
    def k4(xl_hbm, srch_hbm, dsts_hbm, ea_hbm, z128_hbm,
           num_hbm, sidx, didx, eav, gb, acc, sem):
        cid = lax.axis_index("c")
        sid = lax.axis_index("s")
        wid = cid * 16 + sid
        row0 = sid * RT

        pltpu.sync_copy(dsts_hbm.at[wid], didx)

        def head(h, carry):
            pltpu.sync_copy(z128_hbm, acc.at[pl.ds(row0, RT)])
            pltpu.sync_copy(srch_hbm.at[h * NWK + wid], sidx)
            pltpu.sync_copy(ea_hbm.at[h * NWK + wid], eav)
            plsc.subcore_barrier()

            def win(w, carry2):
                pltpu.async_copy(xl_hbm.at[sidx.at[w]], gb, sem).wait()
                for g in range(WIN // 16):
                    evec = eav[w, pl.ds(g * 16, 16)]
                    for jj in range(16):
                        j = g * 16 + jj
                        ev = jnp.full((16,), evec[jj], F32)
                        for q in range(CC // 16):
                            sl = pl.ds(q * 16, 16)
                            gb[j, sl] = gb[j, sl] * ev
                pltpu.sync_copy(gb, acc.at[didx.at[w]], add=True)
                return carry2

            lax.fori_loop(0, NWIN, win, carry)
            plsc.subcore_barrier()
            outbase = (cid * H + h) * RA + row0
            pltpu.sync_copy(acc.at[pl.ds(row0, RT)],
                            num_hbm.at[pl.ds(outbase, RT)])
            return carry

        lax.fori_loop(0, H, head, 0)

    return k4


@functools.lru_cache(maxsize=None)
def _k4d(H):
    mesh = plsc.VectorSubcoreMesh(core_axis_name="c", subcore_axis_name="s")

    @functools.partial(
        pl.kernel,
        out_type=jax.ShapeDtypeStruct((2 * H * RA, CC), F32),
        mesh=mesh,
        scratch_types=[
            pltpu.VMEM((NWIN, WIN), jnp.int32),
            pltpu.VMEM((NWIN, WIN), F32),
            pltpu.VMEM((WIN, CC), F32),
            pltpu.VMEM_SHARED((RA, CC), F32),
        ],
    )
    def k4d(dsts_hbm, ea_hbm, z128_hbm, den_hbm, didx, eav, eb, den):
        cid = lax.axis_index("c")
        sid = lax.axis_index("s")
        wid = cid * 16 + sid
        row0 = sid * RT

        pltpu.sync_copy(dsts_hbm.at[wid], didx)

        def head(h, carry):
            pltpu.sync_copy(z128_hbm, den.at[pl.ds(row0, RT)])
            pltpu.sync_copy(ea_hbm.at[h * NWK + wid], eav)
            plsc.subcore_barrier()

            def win(w, carry2):
                for g in range(WIN // 16):
                    evec = eav[w, pl.ds(g * 16, 16)]
                    for jj in range(16):
                        ev = jnp.full((16,), evec[jj], F32)
                        for q in range(CC // 16):
                            eb[g * 16 + jj, pl.ds(q * 16, 16)] = ev
                pltpu.sync_copy(eb, den.at[didx.at[w]], add=True)
                return carry2

            lax.fori_loop(0, NWIN, win, carry)
            plsc.subcore_barrier()
            outbase = (cid * H + h) * RA + row0
            pltpu.sync_copy(den.at[pl.ds(row0, RT)],
                            den_hbm.at[pl.ds(outbase, RT)])
            return carry

        lax.fori_loop(0, H, head, 0)

    return k4d


def _k5_body(num_ref, den_ref, hp_ref, b_ref, out_ref, *, relu):
    s = num_ref[...].sum(axis=0)
    d = den_ref[...].sum(axis=0)[:, :, 0:1]
    o = (s / (d + 1e-16)).mean(axis=0)
    o = o + b_ref[...]
    if relu:
        o = jnp.maximum(o, 0.0)
    out_ref[...] = o + hp_ref[...]


@functools.lru_cache(maxsize=None)
def _k5(H, relu):
    return pl.pallas_call(
        functools.partial(_k5_body, relu=relu),
        grid=(RA // 128,),
        in_specs=[
            pl.BlockSpec((2, H, 128, CC), lambda nb: (0, 0, nb, 0)),
            pl.BlockSpec((2, H, 128, CC), lambda nb: (0, 0, nb, 0)),
            pl.BlockSpec((128, CC), lambda nb: (nb, 0)),
            pl.BlockSpec((1, CC), lambda nb: (0, 0)),
        ],
        out_specs=pl.BlockSpec((128, CC), lambda nb: (nb, 0)),
        out_shape=jax.ShapeDtypeStruct((NN, CC), F32),
    )


def _layer(h_in, idxs, Wl, Wr, att, b, H, relu):
    srch, dsth, dsts, z128 = idxs[H]
    xl, xr = _k1(H)(h_in, Wl, Wr)
    g = _k2(H)(xl, xr, srch, dsth)
    alpha, gmax = _k3a(H)(g, att)
    ea = _k3b(H)(alpha, gmax)
    ea3 = ea.reshape(H * NWK, NWIN, WIN)
    num = _k4(H)(xl, srch, dsts, ea3, z128)
    den = _k4d(H)(dsts, ea3, z128)
    return _k5(H, relu)(num.reshape(2, H, RA, CC),
                        den.reshape(2, H, RA, CC),
                        h_in, b.reshape(1, CC))


def kernel(x, edge_index, Wl1, Wr1, att1, b1, Wl2, Wr2, att2, b2,
           Wl3, Wr3, att3, b3):
    pad = EP - EE
    loops = jnp.arange(NN, dtype=jnp.int32)
    zpad = jnp.zeros((pad,), jnp.int32)
    src = jnp.concatenate([edge_index[0].astype(jnp.int32), loops, zpad])
    dstg = jnp.concatenate([edge_index[1].astype(jnp.int32), loops, zpad])
    dsts = jnp.concatenate([edge_index[1].astype(jnp.int32), loops,
                            jnp.full((pad,), NN, jnp.int32)])
    dsts = dsts.reshape(NWK, NWIN, WIN)
    z128 = jnp.zeros((RT, CC), F32)

    idxs = {}
    for H in (8, 4):
        offs = jnp.arange(H, dtype=jnp.int32)[:, None] * NN
        srch = (src[None, :] + offs).reshape(H * NWK, NWIN, WIN)
        dsth = (dstg[None, :] + offs).reshape(H * NWK, NWIN, WIN)
        idxs[H] = (srch, dsth, dsts, z128)

    h = _layer(x, idxs, Wl1, Wr1, att1, b1, 8, True)
    h = _layer(h, idxs, Wl2, Wr2, att2, b2, 8, True)
    h = _layer(h, idxs, Wl3, Wr3, att3, b3, 4, False)
    return h


# speedup vs baseline: 4.0685x; 1.0963x over previous
import functools

import jax
import jax.numpy as jnp
from jax import lax
from jax.experimental import pallas as pl
from jax.experimental.pallas import tpu as pltpu
from jax.experimental.pallas import tpu_sc as plsc

NN = 10000
CC = 128
EDGES = 160000
EE = EDGES + NN
NWK = 32
WIN = 128
NWIN = 42
PT = WIN * NWIN
EP = NWK * PT
RA = 10112
RT = RA // 16
BN = 400
NB = NN // BN
EB = 1024
NEB = EP // EB
NEG = 0.2
F32 = jnp.float32


def _k1_body(x_ref, wl_ref, wr_ref, xl_ref, xr_ref):
    x = x_ref[...]
    xl_ref[...] = jnp.dot(x, wl_ref[...], preferred_element_type=F32)
    xr_ref[...] = jnp.dot(x, wr_ref[...], preferred_element_type=F32)


@functools.lru_cache(maxsize=None)
def _k1(H):
    return pl.pallas_call(
        _k1_body,
        grid=(H, NB),
        in_specs=[
            pl.BlockSpec((BN, CC), lambda h, nb: (nb, 0)),
            pl.BlockSpec((CC, CC), lambda h, nb: (0, h)),
            pl.BlockSpec((CC, CC), lambda h, nb: (0, h)),
        ],
        out_specs=[
            pl.BlockSpec((BN, CC), lambda h, nb: (h * NB + nb, 0)),
            pl.BlockSpec((BN, CC), lambda h, nb: (h * NB + nb, 0)),
        ],
        out_shape=[
            jax.ShapeDtypeStruct((H * NN, CC), F32),
            jax.ShapeDtypeStruct((H * NN, CC), F32),
        ],
    )


@functools.lru_cache(maxsize=None)
def _k2(H):
    mesh = plsc.VectorSubcoreMesh(core_axis_name="c", subcore_axis_name="s")

    @functools.partial(
        pl.kernel,
        out_type=jax.ShapeDtypeStruct((H * EP, CC), F32),
        mesh=mesh,
        scratch_types=[
            pltpu.VMEM((NWIN, WIN), jnp.int32),
            pltpu.VMEM((NWIN, WIN), jnp.int32),
            pltpu.VMEM((WIN, CC), F32),
            pltpu.VMEM((WIN, CC), F32),
            pltpu.SemaphoreType.DMA,
            pltpu.SemaphoreType.DMA,
            pltpu.SemaphoreType.DMA,
            pltpu.SemaphoreType.DMA,
            pltpu.SemaphoreType.DMA,
            pltpu.SemaphoreType.DMA,
        ],
    )
    def k2(xl_hbm, xr_hbm, srch_hbm, dsth_hbm, g_hbm, sidx, didx,
           buf0, buf1, sa0, sa1, sb0, sb1, sw0, sw1):
        cid = lax.axis_index("c")
        sid = lax.axis_index("s")
        wid = cid * 16 + sid

        def head(h, carry):
            pltpu.sync_copy(srch_hbm.at[h * NWK + wid], sidx)
            pltpu.sync_copy(dsth_hbm.at[h * NWK + wid], didx)
            base0 = h * EP + wid * PT
            pltpu.async_copy(xl_hbm.at[sidx.at[0]], buf0, sa0)
            pltpu.async_copy(xl_hbm.at[sidx.at[1]], buf1, sa1)

            def pair(t, carry2):
                w0 = 2 * t
                w1 = w0 + 1
                pltpu.make_async_copy(xl_hbm.at[sidx.at[w0]],
                                      buf0, sa0).wait()
                pltpu.async_copy(xr_hbm.at[didx.at[w0]], buf0, sb0,
                                 add=True).wait()
                pltpu.async_copy(buf0, g_hbm.at[pl.ds(base0 + w0 * WIN,
                                                      WIN)], sw0)
                pltpu.make_async_copy(xl_hbm.at[sidx.at[w1]],
                                      buf1, sa1).wait()
                pltpu.async_copy(xr_hbm.at[didx.at[w1]], buf1, sb1,
                                 add=True).wait()
                pltpu.async_copy(buf1, g_hbm.at[pl.ds(base0 + w1 * WIN,
                                                      WIN)], sw1)
                pltpu.make_async_copy(buf0, g_hbm.at[pl.ds(base0, WIN)],
                                      sw0).wait()

                @pl.when(w0 + 2 < NWIN)
                def _():
                    pltpu.async_copy(xl_hbm.at[sidx.at[w0 + 2]], buf0, sa0)

                pltpu.make_async_copy(buf1, g_hbm.at[pl.ds(base0, WIN)],
                                      sw1).wait()

                @pl.when(w1 + 2 < NWIN)
                def _():
                    pltpu.async_copy(xl_hbm.at[sidx.at[w1 + 2]], buf1, sa1)

                return carry2

            lax.fori_loop(0, NWIN // 2, pair, carry)
            return carry

        lax.fori_loop(0, H, head, 0)

    return k2


def _k3a_body(g_ref, att_ref, alpha_ref, gmax_ref):
    eb = pl.program_id(1)
    g = g_ref[...]
    l = jnp.where(g >= 0, g, NEG * g)
    aw = att_ref[pl.program_id(0), :].reshape(1, CC)
    s = (l * aw).sum(axis=1)
    alpha_ref[...] = s

    @pl.when(eb == 0)
    def _():
        gmax_ref[...] = jnp.full((CC,), -jnp.inf, F32)

    gmax_ref[...] = jnp.maximum(gmax_ref[...], jnp.full((CC,), s.max(), F32))


@functools.lru_cache(maxsize=None)
def _k3a(H):
    return pl.pallas_call(
        _k3a_body,
        grid=(H, NEB),
        in_specs=[
            pl.BlockSpec((EB, CC), lambda h, eb: (h * NEB + eb, 0)),
            pl.BlockSpec((H, CC), lambda h, eb: (0, 0)),
        ],
        out_specs=[
            pl.BlockSpec((EB,), lambda h, eb: (h * NEB + eb,)),
            pl.BlockSpec((CC,), lambda h, eb: (h,)),
        ],
        out_shape=[
            jax.ShapeDtypeStruct((H * EP,), F32),
            jax.ShapeDtypeStruct((H * CC,), F32),
        ],
    )


def _k3b_body(alpha_ref, gmax_ref, ea_ref):
    ea_ref[...] = jnp.exp(alpha_ref[...] - gmax_ref[0])


@functools.lru_cache(maxsize=None)
def _k3b(H):
    return pl.pallas_call(
        _k3b_body,
        grid=(H, NEB),
        in_specs=[
            pl.BlockSpec((EB,), lambda h, eb: (h * NEB + eb,)),
            pl.BlockSpec((CC,), lambda h, eb: (h,)),
        ],
        out_specs=pl.BlockSpec((EB,), lambda h, eb: (h * NEB + eb,)),
        out_shape=jax.ShapeDtypeStruct((H * EP,), F32),
    )


@functools.lru_cache(maxsize=None)
def _k4(H):
    mesh = plsc.VectorSubcoreMesh(core_axis_name="c", subcore_axis_name="s")

    @functools.partial(
        pl.kernel,
        out_type=jax.ShapeDtypeStruct((2 * H * RA, CC), F32),
        mesh=mesh,
        scratch_types=[
            pltpu.VMEM((NWIN, WIN), jnp.int32),
            pltpu.VMEM((NWIN, WIN), jnp.int32),
            pltpu.VMEM((NWIN, WIN), F32),
            pltpu.VMEM((WIN, CC), F32),
            pltpu.VMEM_SHARED((RA, CC), F32),
            pltpu.SemaphoreType.DMA,
            pltpu.SemaphoreType.DMA,
        ],
    )
    def k4(xl_hbm, srch_hbm, dsts_hbm, ea_hbm, z128_hbm, num_hbm,
           sidx, didx, eav, gb0, acc, sg0, ss0):
        cid = lax.axis_index("c")
        sid = lax.axis_index("s")
        wid = cid * 16 + sid
        row0 = sid * RT

        pltpu.sync_copy(dsts_hbm.at[wid], didx)

        def scale(gb, w):
            for g in range(WIN // 16):
                evec = eav[w, pl.ds(g * 16, 16)]
                for jj in range(16):
                    j = g * 16 + jj
                    ev = jnp.full((16,), evec[jj], F32)
                    for q in range(CC // 16):
                        sl = pl.ds(q * 16, 16)
                        gb[j, sl] = gb[j, sl] * ev

        def head(h, carry):
            pltpu.sync_copy(z128_hbm, acc.at[pl.ds(row0, RT)])
            pltpu.sync_copy(srch_hbm.at[h * NWK + wid], sidx)
            pltpu.sync_copy(ea_hbm.at[h * NWK + wid], eav)
            plsc.subcore_barrier()

            def win(w, carry2):
                pltpu.async_copy(xl_hbm.at[sidx.at[w]], gb0, sg0).wait()
                scale(gb0, w)
                pltpu.async_copy(gb0, acc.at[didx.at[w]], ss0, add=True).wait()
                return carry2

            lax.fori_loop(0, NWIN, win, carry)
            plsc.subcore_barrier()
            outbase = (cid * H + h) * RA + row0
            pltpu.sync_copy(acc.at[pl.ds(row0, RT)],
                            num_hbm.at[pl.ds(outbase, RT)])
            return carry

        lax.fori_loop(0, H, head, 0)

    return k4


@functools.lru_cache(maxsize=None)
def _k4d(H):
    mesh = plsc.VectorSubcoreMesh(core_axis_name="c", subcore_axis_name="s")

    @functools.partial(
        pl.kernel,
        out_type=jax.ShapeDtypeStruct((2 * H * RA, CC), F32),
        mesh=mesh,
        scratch_types=[
            pltpu.VMEM((NWIN, WIN), jnp.int32),
            pltpu.VMEM((NWIN, WIN), F32),
            pltpu.VMEM((WIN, CC), F32),
            pltpu.VMEM_SHARED((RA, CC), F32),
        ],
    )
    def k4d(dsts_hbm, ea_hbm, z128_hbm, den_hbm, didx, eav, eb, den):
        cid = lax.axis_index("c")
        sid = lax.axis_index("s")
        wid = cid * 16 + sid
        row0 = sid * RT

        pltpu.sync_copy(dsts_hbm.at[wid], didx)

        def head(h, carry):
            pltpu.sync_copy(z128_hbm, den.at[pl.ds(row0, RT)])
            pltpu.sync_copy(ea_hbm.at[h * NWK + wid], eav)
            plsc.subcore_barrier()

            def win(w, carry2):
                for g in range(WIN // 16):
                    evec = eav[w, pl.ds(g * 16, 16)]
                    for jj in range(16):
                        ev = jnp.full((16,), evec[jj], F32)
                        for q in range(CC // 16):
                            eb[g * 16 + jj, pl.ds(q * 16, 16)] = ev
                pltpu.sync_copy(eb, den.at[didx.at[w]], add=True)
                return carry2

            lax.fori_loop(0, NWIN, win, carry)
            plsc.subcore_barrier()
            outbase = (cid * H + h) * RA + row0
            pltpu.sync_copy(den.at[pl.ds(row0, RT)],
                            den_hbm.at[pl.ds(outbase, RT)])
            return carry

        lax.fori_loop(0, H, head, 0)

    return k4d


def _k5_body(num_ref, den_ref, hp_ref, b_ref, out_ref, *, relu):
    s = num_ref[...].sum(axis=0)
    d = den_ref[...].sum(axis=0)[:, :, 0:1]
    o = (s / (d + 1e-16)).mean(axis=0)
    o = o + b_ref[...]
    if relu:
        o = jnp.maximum(o, 0.0)
    out_ref[...] = o + hp_ref[...]


@functools.lru_cache(maxsize=None)
def _k5(H, relu):
    return pl.pallas_call(
        functools.partial(_k5_body, relu=relu),
        grid=(RA // 128,),
        in_specs=[
            pl.BlockSpec((2, H, 128, CC), lambda nb: (0, 0, nb, 0)),
            pl.BlockSpec((2, H, 128, CC), lambda nb: (0, 0, nb, 0)),
            pl.BlockSpec((128, CC), lambda nb: (nb, 0)),
            pl.BlockSpec((1, CC), lambda nb: (0, 0)),
        ],
        out_specs=pl.BlockSpec((128, CC), lambda nb: (nb, 0)),
        out_shape=jax.ShapeDtypeStruct((NN, CC), F32),
    )


def _layer(h_in, idxs, Wl, Wr, att, b, H, relu):
    srch, dsth, dsts, z128 = idxs[H]
    xl, xr = _k1(H)(h_in, Wl, Wr)
    g = _k2(H)(xl, xr, srch, dsth)
    alpha, gmax = _k3a(H)(g, att)
    ea = _k3b(H)(alpha, gmax)
    ea3 = ea.reshape(H * NWK, NWIN, WIN)
    num = _k4(H)(xl, srch, dsts, ea3, z128)
    den = _k4d(H)(dsts, ea3, z128)
    return _k5(H, relu)(num.reshape(2, H, RA, CC),
                        den.reshape(2, H, RA, CC),
                        h_in, b.reshape(1, CC))


def kernel(x, edge_index, Wl1, Wr1, att1, b1, Wl2, Wr2, att2, b2,
           Wl3, Wr3, att3, b3):
    pad = EP - EE
    loops = jnp.arange(NN, dtype=jnp.int32)
    zpad = jnp.zeros((pad,), jnp.int32)
    src = jnp.concatenate([edge_index[0].astype(jnp.int32), loops, zpad])
    dstg = jnp.concatenate([edge_index[1].astype(jnp.int32), loops, zpad])
    dsts = jnp.concatenate([edge_index[1].astype(jnp.int32), loops,
                            jnp.full((pad,), NN, jnp.int32)])
    dsts = dsts.reshape(NWK, NWIN, WIN)
    z128 = jnp.zeros((RT, CC), F32)

    idxs = {}
    for H in (8, 4):
        offs = jnp.arange(H, dtype=jnp.int32)[:, None] * NN
        srch = (src[None, :] + offs).reshape(H * NWK, NWIN, WIN)
        dsth = (dstg[None, :] + offs).reshape(H * NWK, NWIN, WIN)
        idxs[H] = (srch, dsth, dsts, z128)

    h = _layer(x, idxs, Wl1, Wr1, att1, b1, 8, True)
    h = _layer(h, idxs, Wl2, Wr2, att2, b2, 8, True)
    h = _layer(h, idxs, Wl3, Wr3, att3, b3, 4, False)
    return h


# speedup vs baseline: 4.3612x; 1.0719x over previous
import functools

import jax
import jax.numpy as jnp
from jax import lax
from jax.experimental import pallas as pl
from jax.experimental.pallas import tpu as pltpu
from jax.experimental.pallas import tpu_sc as plsc

NN = 10000
CC = 128
EDGES = 160000
EE = EDGES + NN
NWK = 32
WIN = 128
NWIN = 42
W4 = 64
NW4 = 84
PT = WIN * NWIN
EP = NWK * PT
RA = 10112
RT = RA // 16
BN = 400
NB = NN // BN
EB = 1024
NEB = EP // EB
NEG = 0.2
F32 = jnp.float32


def _k1_body(x_ref, wl_ref, wr_ref, xl_ref, xr_ref):
    x = x_ref[...]
    xl_ref[...] = jnp.dot(x, wl_ref[...], preferred_element_type=F32)
    xr_ref[...] = jnp.dot(x, wr_ref[...], preferred_element_type=F32)


@functools.lru_cache(maxsize=None)
def _k1(H):
    return pl.pallas_call(
        _k1_body,
        grid=(H, NB),
        in_specs=[
            pl.BlockSpec((BN, CC), lambda h, nb: (nb, 0)),
            pl.BlockSpec((CC, CC), lambda h, nb: (0, h)),
            pl.BlockSpec((CC, CC), lambda h, nb: (0, h)),
        ],
        out_specs=[
            pl.BlockSpec((BN, CC), lambda h, nb: (h * NB + nb, 0)),
            pl.BlockSpec((BN, CC), lambda h, nb: (h * NB + nb, 0)),
        ],
        out_shape=[
            jax.ShapeDtypeStruct((H * NN, CC), F32),
            jax.ShapeDtypeStruct((H * NN, CC), F32),
        ],
    )


@functools.lru_cache(maxsize=None)
def _k2(H):
    mesh = plsc.VectorSubcoreMesh(core_axis_name="c", subcore_axis_name="s")

    @functools.partial(
        pl.kernel,
        out_type=jax.ShapeDtypeStruct((H * EP, CC), F32),
        mesh=mesh,
        scratch_types=[
            pltpu.VMEM((NWIN, WIN), jnp.int32),
            pltpu.VMEM((NWIN, WIN), jnp.int32),
            pltpu.VMEM((WIN, CC), F32),
            pltpu.VMEM((WIN, CC), F32),
            pltpu.SemaphoreType.DMA,
            pltpu.SemaphoreType.DMA,
            pltpu.SemaphoreType.DMA,
            pltpu.SemaphoreType.DMA,
            pltpu.SemaphoreType.DMA,
            pltpu.SemaphoreType.DMA,
        ],
    )
    def k2(xl_hbm, xr_hbm, srch_hbm, dsth_hbm, g_hbm, sidx, didx,
           buf0, buf1, sa0, sa1, sb0, sb1, sw0, sw1):
        cid = lax.axis_index("c")
        sid = lax.axis_index("s")
        wid = cid * 16 + sid

        def head(h, carry):
            pltpu.sync_copy(srch_hbm.at[h * NWK + wid], sidx)
            pltpu.sync_copy(dsth_hbm.at[h * NWK + wid], didx)
            base0 = h * EP + wid * PT
            pltpu.async_copy(xl_hbm.at[sidx.at[0]], buf0, sa0)
            pltpu.async_copy(xl_hbm.at[sidx.at[1]], buf1, sa1)

            def pair(t, carry2):
                w0 = 2 * t
                w1 = w0 + 1
                pltpu.make_async_copy(xl_hbm.at[sidx.at[w0]],
                                      buf0, sa0).wait()
                pltpu.async_copy(xr_hbm.at[didx.at[w0]], buf0, sb0,
                                 add=True).wait()
                pltpu.async_copy(buf0, g_hbm.at[pl.ds(base0 + w0 * WIN,
                                                      WIN)], sw0)
                pltpu.make_async_copy(xl_hbm.at[sidx.at[w1]],
                                      buf1, sa1).wait()
                pltpu.async_copy(xr_hbm.at[didx.at[w1]], buf1, sb1,
                                 add=True).wait()
                pltpu.async_copy(buf1, g_hbm.at[pl.ds(base0 + w1 * WIN,
                                                      WIN)], sw1)
                pltpu.make_async_copy(buf0, g_hbm.at[pl.ds(base0, WIN)],
                                      sw0).wait()

                @pl.when(w0 + 2 < NWIN)
                def _():
                    pltpu.async_copy(xl_hbm.at[sidx.at[w0 + 2]], buf0, sa0)

                pltpu.make_async_copy(buf1, g_hbm.at[pl.ds(base0, WIN)],
                                      sw1).wait()

                @pl.when(w1 + 2 < NWIN)
                def _():
                    pltpu.async_copy(xl_hbm.at[sidx.at[w1 + 2]], buf1, sa1)

                return carry2

            lax.fori_loop(0, NWIN // 2, pair, carry)
            return carry

        lax.fori_loop(0, H, head, 0)

    return k2


def _k3a_body(g_ref, att_ref, alpha_ref, gmax_ref):
    eb = pl.program_id(1)
    g = g_ref[...]
    l = jnp.where(g >= 0, g, NEG * g)
    aw = att_ref[pl.program_id(0), :].reshape(1, CC)
    s = (l * aw).sum(axis=1)
    alpha_ref[...] = s

    @pl.when(eb == 0)
    def _():
        gmax_ref[...] = jnp.full((CC,), -jnp.inf, F32)

    gmax_ref[...] = jnp.maximum(gmax_ref[...], jnp.full((CC,), s.max(), F32))


@functools.lru_cache(maxsize=None)
def _k3a(H):
    return pl.pallas_call(
        _k3a_body,
        grid=(H, NEB),
        in_specs=[
            pl.BlockSpec((EB, CC), lambda h, eb: (h * NEB + eb, 0)),
            pl.BlockSpec((H, CC), lambda h, eb: (0, 0)),
        ],
        out_specs=[
            pl.BlockSpec((EB,), lambda h, eb: (h * NEB + eb,)),
            pl.BlockSpec((CC,), lambda h, eb: (h,)),
        ],
        out_shape=[
            jax.ShapeDtypeStruct((H * EP,), F32),
            jax.ShapeDtypeStruct((H * CC,), F32),
        ],
    )


def _k3b_body(alpha_ref, gmax_ref, ea_ref):
    ea_ref[...] = jnp.exp(alpha_ref[...] - gmax_ref[0])


@functools.lru_cache(maxsize=None)
def _k3b(H):
    return pl.pallas_call(
        _k3b_body,
        grid=(H, NEB),
        in_specs=[
            pl.BlockSpec((EB,), lambda h, eb: (h * NEB + eb,)),
            pl.BlockSpec((CC,), lambda h, eb: (h,)),
        ],
        out_specs=pl.BlockSpec((EB,), lambda h, eb: (h * NEB + eb,)),
        out_shape=jax.ShapeDtypeStruct((H * EP,), F32),
    )


@functools.lru_cache(maxsize=None)
def _k4(H):
    mesh = plsc.VectorSubcoreMesh(core_axis_name="c", subcore_axis_name="s")

    @functools.partial(
        pl.kernel,
        out_type=jax.ShapeDtypeStruct((2 * H * RA, CC), F32),
        mesh=mesh,
        scratch_types=[
            pltpu.VMEM((NW4, W4), jnp.int32),
            pltpu.VMEM((NW4, W4), jnp.int32),
            pltpu.VMEM((NW4, W4), F32),
            pltpu.VMEM((W4, CC), F32),
            pltpu.VMEM((W4, CC), F32),
            pltpu.VMEM_SHARED((RA, CC), F32),
            pltpu.SemaphoreType.DMA,
            pltpu.SemaphoreType.DMA,
            pltpu.SemaphoreType.DMA,
            pltpu.SemaphoreType.DMA,
        ],
    )
    def k4(xl_hbm, srch_hbm, dsts_hbm, ea_hbm, z128_hbm, num_hbm,
           sidx, didx, eav, gb0, gb1, acc, sg0, sg1, ss0, ss1):
        cid = lax.axis_index("c")
        sid = lax.axis_index("s")
        wid = cid * 16 + sid
        row0 = sid * RT

        pltpu.sync_copy(dsts_hbm.at[wid], didx)

        def scale(gb, w):
            for g in range(W4 // 16):
                evec = eav[w, pl.ds(g * 16, 16)]
                for jj in range(16):
                    j = g * 16 + jj
                    ev = jnp.full((16,), evec[jj], F32)
                    for q in range(CC // 16):
                        sl = pl.ds(q * 16, 16)
                        gb[j, sl] = gb[j, sl] * ev

        def head(h, carry):
            pltpu.sync_copy(z128_hbm, acc.at[pl.ds(row0, RT)])
            pltpu.sync_copy(srch_hbm.at[h * NWK + wid], sidx)
            pltpu.sync_copy(ea_hbm.at[h * NWK + wid], eav)
            plsc.subcore_barrier()

            pltpu.async_copy(xl_hbm.at[sidx.at[0]], gb0, sg0)
            pltpu.async_copy(xl_hbm.at[sidx.at[1]], gb1, sg1)

            def pair(t, carry2):
                w0 = 2 * t
                w1 = w0 + 1
                pltpu.make_async_copy(xl_hbm.at[sidx.at[w0]], gb0, sg0).wait()
                scale(gb0, w0)
                pltpu.async_copy(gb0, acc.at[didx.at[w0]], ss0, add=True)
                pltpu.make_async_copy(xl_hbm.at[sidx.at[w1]], gb1, sg1).wait()
                scale(gb1, w1)
                pltpu.async_copy(gb1, acc.at[didx.at[w1]], ss1, add=True)
                pltpu.make_async_copy(gb0, acc.at[didx.at[w0]], ss0).wait()

                @pl.when(w0 + 2 < NW4)
                def _():
                    pltpu.async_copy(xl_hbm.at[sidx.at[w0 + 2]], gb0, sg0)

                pltpu.make_async_copy(gb1, acc.at[didx.at[w1]], ss1).wait()

                @pl.when(w1 + 2 < NW4)
                def _():
                    pltpu.async_copy(xl_hbm.at[sidx.at[w1 + 2]], gb1, sg1)

                return carry2

            lax.fori_loop(0, NW4 // 2, pair, carry)
            plsc.subcore_barrier()
            outbase = (cid * H + h) * RA + row0
            pltpu.sync_copy(acc.at[pl.ds(row0, RT)],
                            num_hbm.at[pl.ds(outbase, RT)])
            return carry

        lax.fori_loop(0, H, head, 0)

    return k4


@functools.lru_cache(maxsize=None)
def _k4d(H):
    mesh = plsc.VectorSubcoreMesh(core_axis_name="c", subcore_axis_name="s")

    @functools.partial(
        pl.kernel,
        out_type=jax.ShapeDtypeStruct((2 * H * RA, CC), F32),
        mesh=mesh,
        scratch_types=[
            pltpu.VMEM((NW4, W4), jnp.int32),
            pltpu.VMEM((NW4, W4), F32),
            pltpu.VMEM((W4, CC), F32),
            pltpu.VMEM((W4, CC), F32),
            pltpu.VMEM_SHARED((RA, CC), F32),
            pltpu.SemaphoreType.DMA,
            pltpu.SemaphoreType.DMA,
        ],
    )
    def k4d(dsts_hbm, ea_hbm, z128_hbm, den_hbm,
            didx, eav, eb0, eb1, den, ss0, ss1):
        cid = lax.axis_index("c")
        sid = lax.axis_index("s")
        wid = cid * 16 + sid
        row0 = sid * RT

        pltpu.sync_copy(dsts_hbm.at[wid], didx)

        def fill(eb, w):
            for g in range(W4 // 16):
                evec = eav[w, pl.ds(g * 16, 16)]
                for jj in range(16):
                    ev = jnp.full((16,), evec[jj], F32)
                    for q in range(CC // 16):
                        eb[g * 16 + jj, pl.ds(q * 16, 16)] = ev

        def head(h, carry):
            pltpu.sync_copy(z128_hbm, den.at[pl.ds(row0, RT)])
            pltpu.sync_copy(ea_hbm.at[h * NWK + wid], eav)
            plsc.subcore_barrier()

            def pair(t, carry2):
                w0 = 2 * t
                w1 = w0 + 1

                @pl.when(t > 0)
                def _():
                    pltpu.make_async_copy(eb0, den.at[didx.at[w0]],
                                          ss0).wait()

                fill(eb0, w0)
                pltpu.async_copy(eb0, den.at[didx.at[w0]], ss0, add=True)

                @pl.when(t > 0)
                def _():
                    pltpu.make_async_copy(eb1, den.at[didx.at[w1]],
                                          ss1).wait()

                fill(eb1, w1)
                pltpu.async_copy(eb1, den.at[didx.at[w1]], ss1, add=True)
                return carry2

            lax.fori_loop(0, NW4 // 2, pair, carry)
            pltpu.make_async_copy(eb0, den.at[didx.at[0]], ss0).wait()
            pltpu.make_async_copy(eb1, den.at[didx.at[0]], ss1).wait()
            plsc.subcore_barrier()
            outbase = (cid * H + h) * RA + row0
            pltpu.sync_copy(den.at[pl.ds(row0, RT)],
                            den_hbm.at[pl.ds(outbase, RT)])
            return carry

        lax.fori_loop(0, H, head, 0)

    return k4d


def _k5_body(num_ref, den_ref, hp_ref, b_ref, out_ref, *, relu):
    s = num_ref[...].sum(axis=0)
    d = den_ref[...].sum(axis=0)[:, :, 0:1]
    o = (s / (d + 1e-16)).mean(axis=0)
    o = o + b_ref[...]
    if relu:
        o = jnp.maximum(o, 0.0)
    out_ref[...] = o + hp_ref[...]


@functools.lru_cache(maxsize=None)
def _k5(H, relu):
    return pl.pallas_call(
        functools.partial(_k5_body, relu=relu),
        grid=(RA // 128,),
        in_specs=[
            pl.BlockSpec((2, H, 128, CC), lambda nb: (0, 0, nb, 0)),
            pl.BlockSpec((2, H, 128, CC), lambda nb: (0, 0, nb, 0)),
            pl.BlockSpec((128, CC), lambda nb: (nb, 0)),
            pl.BlockSpec((1, CC), lambda nb: (0, 0)),
        ],
        out_specs=pl.BlockSpec((128, CC), lambda nb: (nb, 0)),
        out_shape=jax.ShapeDtypeStruct((NN, CC), F32),
    )


def _layer(h_in, idxs, Wl, Wr, att, b, H, relu):
    srch, dsth, dsts, z128 = idxs[H]
    xl, xr = _k1(H)(h_in, Wl, Wr)
    g = _k2(H)(xl, xr, srch, dsth)
    alpha, gmax = _k3a(H)(g, att)
    ea = _k3b(H)(alpha, gmax)
    ea4 = ea.reshape(H * NWK, NW4, W4)
    srch4 = srch.reshape(H * NWK, NW4, W4)
    dsts4 = dsts.reshape(NWK, NW4, W4)
    num = _k4(H)(xl, srch4, dsts4, ea4, z128)
    den = _k4d(H)(dsts4, ea4, z128)
    return _k5(H, relu)(num.reshape(2, H, RA, CC),
                        den.reshape(2, H, RA, CC),
                        h_in, b.reshape(1, CC))


def kernel(x, edge_index, Wl1, Wr1, att1, b1, Wl2, Wr2, att2, b2,
           Wl3, Wr3, att3, b3):
    pad = EP - EE
    loops = jnp.arange(NN, dtype=jnp.int32)
    zpad = jnp.zeros((pad,), jnp.int32)
    src = jnp.concatenate([edge_index[0].astype(jnp.int32), loops, zpad])
    dstg = jnp.concatenate([edge_index[1].astype(jnp.int32), loops, zpad])
    dsts = jnp.concatenate([edge_index[1].astype(jnp.int32), loops,
                            jnp.full((pad,), NN, jnp.int32)])
    dsts = dsts.reshape(NWK, NWIN, WIN)
    z128 = jnp.zeros((RT, CC), F32)

    idxs = {}
    for H in (8, 4):
        offs = jnp.arange(H, dtype=jnp.int32)[:, None] * NN
        srch = (src[None, :] + offs).reshape(H * NWK, NWIN, WIN)
        dsth = (dstg[None, :] + offs).reshape(H * NWK, NWIN, WIN)
        idxs[H] = (srch, dsth, dsts, z128)

    h = _layer(x, idxs, Wl1, Wr1, att1, b1, 8, True)
    h = _layer(h, idxs, Wl2, Wr2, att2, b2, 8, True)
    h = _layer(h, idxs, Wl3, Wr3, att3, b3, 4, False)
    return h


# speedup vs baseline: 4.3829x; 1.0050x over previous
import functools

import jax
import jax.numpy as jnp
from jax import lax
from jax.experimental import pallas as pl
from jax.experimental.pallas import tpu as pltpu
from jax.experimental.pallas import tpu_sc as plsc

NN = 10000
CC = 128
EDGES = 160000
EE = EDGES + NN
NWK = 32
WIN = 128
NWIN = 42
W4 = 64
NW4 = 84
PT = WIN * NWIN
EP = NWK * PT
RA = 10112
DW = 128
RT = RA // 16
BN = 400
NB = NN // BN
EB = 1024
NEB = EP // EB
NEG = 0.2
F32 = jnp.float32


def _k1_body(x_ref, wl_ref, wr_ref, xl_ref, xr_ref):
    x = x_ref[...]
    xl_ref[...] = jnp.dot(x, wl_ref[...], preferred_element_type=F32)
    xr_ref[...] = jnp.dot(x, wr_ref[...], preferred_element_type=F32)


@functools.lru_cache(maxsize=None)
def _k1(H):
    return pl.pallas_call(
        _k1_body,
        grid=(H, NB),
        in_specs=[
            pl.BlockSpec((BN, CC), lambda h, nb: (nb, 0)),
            pl.BlockSpec((CC, CC), lambda h, nb: (0, h)),
            pl.BlockSpec((CC, CC), lambda h, nb: (0, h)),
        ],
        out_specs=[
            pl.BlockSpec((BN, CC), lambda h, nb: (h * NB + nb, 0)),
            pl.BlockSpec((BN, CC), lambda h, nb: (h * NB + nb, 0)),
        ],
        out_shape=[
            jax.ShapeDtypeStruct((H * NN, CC), F32),
            jax.ShapeDtypeStruct((H * NN, CC), F32),
        ],
    )


@functools.lru_cache(maxsize=None)
def _k2(H):
    mesh = plsc.VectorSubcoreMesh(core_axis_name="c", subcore_axis_name="s")

    @functools.partial(
        pl.kernel,
        out_type=jax.ShapeDtypeStruct((H * EP, 16), F32),
        mesh=mesh,
        scratch_types=[
            pltpu.VMEM((NWIN, WIN), jnp.int32),
            pltpu.VMEM((NWIN, WIN), jnp.int32),
            pltpu.VMEM((WIN, CC), F32),
            pltpu.VMEM((WIN, CC), F32),
            pltpu.VMEM((WIN, 16), F32),
            pltpu.VMEM((WIN, 16), F32),
            pltpu.VMEM((CC,), F32),
            pltpu.SemaphoreType.DMA,
            pltpu.SemaphoreType.DMA,
            pltpu.SemaphoreType.DMA,
            pltpu.SemaphoreType.DMA,
            pltpu.SemaphoreType.DMA,
            pltpu.SemaphoreType.DMA,
        ],
    )
    def k2(xl_hbm, xr_hbm, srch_hbm, dsth_hbm, att_hbm, ap_hbm, sidx, didx,
           buf0, buf1, ap0, ap1, attv, sa0, sa1, sb0, sb1, sw0, sw1):
        cid = lax.axis_index("c")
        sid = lax.axis_index("s")
        wid = cid * 16 + sid

        def head(h, carry):
            pltpu.sync_copy(srch_hbm.at[h * NWK + wid], sidx)
            pltpu.sync_copy(dsth_hbm.at[h * NWK + wid], didx)
            pltpu.sync_copy(att_hbm.at[h], attv)
            aq = [attv[pl.ds(q * 16, 16)] for q in range(CC // 16)]
            base0 = h * EP + wid * PT
            pltpu.async_copy(xl_hbm.at[sidx.at[0]], buf0, sa0)
            pltpu.async_copy(xl_hbm.at[sidx.at[1]], buf1, sa1)

            def alpha_partial(buf, ap):
                for j in range(WIN):
                    acc = None
                    for q in range(CC // 16):
                        v = buf[j, pl.ds(q * 16, 16)]
                        lr = jnp.where(v >= 0, v, NEG * v)
                        t = aq[q] * lr
                        acc = t if acc is None else acc + t
                    ap[j, :] = acc

            def pair(t, carry2):
                w0 = 2 * t
                w1 = w0 + 1
                pltpu.make_async_copy(xl_hbm.at[sidx.at[w0]],
                                      buf0, sa0).wait()
                pltpu.async_copy(xr_hbm.at[didx.at[w0]], buf0, sb0,
                                 add=True).wait()

                @pl.when(t > 0)
                def _():
                    pltpu.make_async_copy(
                        ap0, ap_hbm.at[pl.ds(base0, WIN)], sw0).wait()

                alpha_partial(buf0, ap0)
                pltpu.async_copy(ap0, ap_hbm.at[pl.ds(base0 + w0 * WIN,
                                                      WIN)], sw0)

                @pl.when(w0 + 2 < NWIN)
                def _():
                    pltpu.async_copy(xl_hbm.at[sidx.at[w0 + 2]], buf0, sa0)

                pltpu.make_async_copy(xl_hbm.at[sidx.at[w1]],
                                      buf1, sa1).wait()
                pltpu.async_copy(xr_hbm.at[didx.at[w1]], buf1, sb1,
                                 add=True).wait()

                @pl.when(t > 0)
                def _():
                    pltpu.make_async_copy(
                        ap1, ap_hbm.at[pl.ds(base0, WIN)], sw1).wait()

                alpha_partial(buf1, ap1)
                pltpu.async_copy(ap1, ap_hbm.at[pl.ds(base0 + w1 * WIN,
                                                      WIN)], sw1)

                @pl.when(w1 + 2 < NWIN)
                def _():
                    pltpu.async_copy(xl_hbm.at[sidx.at[w1 + 2]], buf1, sa1)

                return carry2

            lax.fori_loop(0, NWIN // 2, pair, carry)
            pltpu.make_async_copy(ap0, ap_hbm.at[pl.ds(base0, WIN)],
                                  sw0).wait()
            pltpu.make_async_copy(ap1, ap_hbm.at[pl.ds(base0, WIN)],
                                  sw1).wait()
            return carry

        lax.fori_loop(0, H, head, 0)

    return k2


def _k3a_body(ap_ref, alpha_ref, gmax_ref):
    eb = pl.program_id(1)
    s = ap_ref[...].sum(axis=1)
    alpha_ref[...] = s

    @pl.when(eb == 0)
    def _():
        gmax_ref[...] = jnp.full((CC,), -jnp.inf, F32)

    gmax_ref[...] = jnp.maximum(gmax_ref[...], jnp.full((CC,), s.max(), F32))


@functools.lru_cache(maxsize=None)
def _k3a(H):
    return pl.pallas_call(
        _k3a_body,
        grid=(H, NEB),
        in_specs=[
            pl.BlockSpec((EB, 16), lambda h, eb: (h * NEB + eb, 0)),
        ],
        out_specs=[
            pl.BlockSpec((EB,), lambda h, eb: (h * NEB + eb,)),
            pl.BlockSpec((CC,), lambda h, eb: (h,)),
        ],
        out_shape=[
            jax.ShapeDtypeStruct((H * EP,), F32),
            jax.ShapeDtypeStruct((H * CC,), F32),
        ],
    )


def _k3b_body(alpha_ref, gmax_ref, ea_ref):
    ea_ref[...] = jnp.exp(alpha_ref[...] - gmax_ref[0])


@functools.lru_cache(maxsize=None)
def _k3b(H):
    return pl.pallas_call(
        _k3b_body,
        grid=(H, NEB),
        in_specs=[
            pl.BlockSpec((EB,), lambda h, eb: (h * NEB + eb,)),
            pl.BlockSpec((CC,), lambda h, eb: (h,)),
        ],
        out_specs=pl.BlockSpec((EB,), lambda h, eb: (h * NEB + eb,)),
        out_shape=jax.ShapeDtypeStruct((H * EP,), F32),
    )


@functools.lru_cache(maxsize=None)
def _k4(H):
    mesh = plsc.VectorSubcoreMesh(core_axis_name="c", subcore_axis_name="s")

    @functools.partial(
        pl.kernel,
        out_type=jax.ShapeDtypeStruct((2 * H * RA, CC), F32),
        mesh=mesh,
        scratch_types=[
            pltpu.VMEM((NW4, W4), jnp.int32),
            pltpu.VMEM((NW4, W4), jnp.int32),
            pltpu.VMEM((NW4, W4), F32),
            pltpu.VMEM((W4, CC), F32),
            pltpu.VMEM((W4, CC), F32),
            pltpu.VMEM_SHARED((RA, CC), F32),
            pltpu.SemaphoreType.DMA,
            pltpu.SemaphoreType.DMA,
            pltpu.SemaphoreType.DMA,
            pltpu.SemaphoreType.DMA,
        ],
    )
    def k4(xl_hbm, srch_hbm, dsts_hbm, ea_hbm, z128_hbm, num_hbm,
           sidx, didx, eav, gb0, gb1, acc, sg0, sg1, ss0, ss1):
        cid = lax.axis_index("c")
        sid = lax.axis_index("s")
        wid = cid * 16 + sid
        row0 = sid * RT

        pltpu.sync_copy(dsts_hbm.at[wid], didx)

        def scale(gb, w):
            for g in range(W4 // 16):
                evec = eav[w, pl.ds(g * 16, 16)]
                for jj in range(16):
                    j = g * 16 + jj
                    ev = jnp.full((16,), evec[jj], F32)
                    for q in range(CC // 16):
                        sl = pl.ds(q * 16, 16)
                        gb[j, sl] = gb[j, sl] * ev

        def head(h, carry):
            pltpu.sync_copy(z128_hbm, acc.at[pl.ds(row0, RT)])
            pltpu.sync_copy(srch_hbm.at[h * NWK + wid], sidx)
            pltpu.sync_copy(ea_hbm.at[h * NWK + wid], eav)
            plsc.subcore_barrier()

            pltpu.async_copy(xl_hbm.at[sidx.at[0]], gb0, sg0)
            pltpu.async_copy(xl_hbm.at[sidx.at[1]], gb1, sg1)

            def pair(t, carry2):
                w0 = 2 * t
                w1 = w0 + 1
                pltpu.make_async_copy(xl_hbm.at[sidx.at[w0]], gb0, sg0).wait()
                scale(gb0, w0)
                pltpu.async_copy(gb0, acc.at[didx.at[w0]], ss0, add=True)
                pltpu.make_async_copy(xl_hbm.at[sidx.at[w1]], gb1, sg1).wait()
                scale(gb1, w1)
                pltpu.async_copy(gb1, acc.at[didx.at[w1]], ss1, add=True)
                pltpu.make_async_copy(gb0, acc.at[didx.at[w0]], ss0).wait()

                @pl.when(w0 + 2 < NW4)
                def _():
                    pltpu.async_copy(xl_hbm.at[sidx.at[w0 + 2]], gb0, sg0)

                pltpu.make_async_copy(gb1, acc.at[didx.at[w1]], ss1).wait()

                @pl.when(w1 + 2 < NW4)
                def _():
                    pltpu.async_copy(xl_hbm.at[sidx.at[w1 + 2]], gb1, sg1)

                return carry2

            lax.fori_loop(0, NW4 // 2, pair, carry)
            plsc.subcore_barrier()
            outbase = (cid * H + h) * RA + row0
            pltpu.sync_copy(acc.at[pl.ds(row0, RT)],
                            num_hbm.at[pl.ds(outbase, RT)])
            return carry

        lax.fori_loop(0, H, head, 0)

    return k4


@functools.lru_cache(maxsize=None)
def _k4d(H):
    mesh = plsc.VectorSubcoreMesh(core_axis_name="c", subcore_axis_name="s")

    @functools.partial(
        pl.kernel,
        out_type=jax.ShapeDtypeStruct((2 * H * RA, DW), F32),
        mesh=mesh,
        scratch_types=[
            pltpu.VMEM((NW4, W4), jnp.int32),
            pltpu.VMEM((NW4, W4), F32),
            pltpu.VMEM((W4, DW), F32),
            pltpu.VMEM((W4, DW), F32),
            pltpu.VMEM_SHARED((RA, DW), F32),
            pltpu.SemaphoreType.DMA,
            pltpu.SemaphoreType.DMA,
        ],
    )
    def k4d(dsts_hbm, ea_hbm, zden_hbm, den_hbm,
            didx, eav, eb0, eb1, den, ss0, ss1):
        cid = lax.axis_index("c")
        sid = lax.axis_index("s")
        wid = cid * 16 + sid
        row0 = sid * RT

        pltpu.sync_copy(dsts_hbm.at[wid], didx)

        def fill(eb, w):
            for g in range(W4 // 16):
                evec = eav[w, pl.ds(g * 16, 16)]
                for jj in range(16):
                    ev = jnp.full((16,), evec[jj], F32)
                    for q in range(DW // 16):
                        eb[g * 16 + jj, pl.ds(q * 16, 16)] = ev

        def head(h, carry):
            pltpu.sync_copy(zden_hbm, den.at[pl.ds(row0, RT)])
            pltpu.sync_copy(ea_hbm.at[h * NWK + wid], eav)
            plsc.subcore_barrier()

            def pair(t, carry2):
                w0 = 2 * t
                w1 = w0 + 1

                @pl.when(t > 0)
                def _():
                    pltpu.make_async_copy(eb0, den.at[didx.at[w0]],
                                          ss0).wait()

                fill(eb0, w0)
                pltpu.async_copy(eb0, den.at[didx.at[w0]], ss0, add=True)

                @pl.when(t > 0)
                def _():
                    pltpu.make_async_copy(eb1, den.at[didx.at[w1]],
                                          ss1).wait()

                fill(eb1, w1)
                pltpu.async_copy(eb1, den.at[didx.at[w1]], ss1, add=True)
                return carry2

            lax.fori_loop(0, NW4 // 2, pair, carry)
            pltpu.make_async_copy(eb0, den.at[didx.at[0]], ss0).wait()
            pltpu.make_async_copy(eb1, den.at[didx.at[0]], ss1).wait()
            plsc.subcore_barrier()
            outbase = (cid * H + h) * RA + row0
            pltpu.sync_copy(den.at[pl.ds(row0, RT)],
                            den_hbm.at[pl.ds(outbase, RT)])
            return carry

        lax.fori_loop(0, H, head, 0)

    return k4d


def _k5_body(num_ref, den_ref, hp_ref, b_ref, out_ref, *, relu):
    s = num_ref[...].sum(axis=0)
    d = den_ref[...].sum(axis=0)[:, :, 0:1]
    o = (s / (d + 1e-16)).mean(axis=0)
    o = o + b_ref[...]
    if relu:
        o = jnp.maximum(o, 0.0)
    out_ref[...] = o + hp_ref[...]


@functools.lru_cache(maxsize=None)
def _k5(H, relu):
    return pl.pallas_call(
        functools.partial(_k5_body, relu=relu),
        grid=(RA // 128,),
        in_specs=[
            pl.BlockSpec((2, H, 128, CC), lambda nb: (0, 0, nb, 0)),
            pl.BlockSpec((2, H, 128, DW), lambda nb: (0, 0, nb, 0)),
            pl.BlockSpec((128, CC), lambda nb: (nb, 0)),
            pl.BlockSpec((1, CC), lambda nb: (0, 0)),
        ],
        out_specs=pl.BlockSpec((128, CC), lambda nb: (nb, 0)),
        out_shape=jax.ShapeDtypeStruct((NN, CC), F32),
    )


def _layer(h_in, idxs, Wl, Wr, att, b, H, relu):
    srch, dsth, dsts, z128, zden = idxs[H]
    xl, xr = _k1(H)(h_in, Wl, Wr)
    ap = _k2(H)(xl, xr, srch, dsth, att)
    alpha, gmax = _k3a(H)(ap)
    ea = _k3b(H)(alpha, gmax)
    ea4 = ea.reshape(H * NWK, NW4, W4)
    srch4 = srch.reshape(H * NWK, NW4, W4)
    dsts4 = dsts.reshape(NWK, NW4, W4)
    num = _k4(H)(xl, srch4, dsts4, ea4, z128)
    den = _k4d(H)(dsts4, ea4, zden)
    return _k5(H, relu)(num.reshape(2, H, RA, CC),
                        den.reshape(2, H, RA, DW),
                        h_in, b.reshape(1, CC))


def kernel(x, edge_index, Wl1, Wr1, att1, b1, Wl2, Wr2, att2, b2,
           Wl3, Wr3, att3, b3):
    pad = EP - EE
    loops = jnp.arange(NN, dtype=jnp.int32)
    zpad = jnp.zeros((pad,), jnp.int32)
    src = jnp.concatenate([edge_index[0].astype(jnp.int32), loops, zpad])
    dstg = jnp.concatenate([edge_index[1].astype(jnp.int32), loops, zpad])
    dsts = jnp.concatenate([edge_index[1].astype(jnp.int32), loops,
                            jnp.full((pad,), NN, jnp.int32)])
    dsts = dsts.reshape(NWK, NWIN, WIN)
    z128 = jnp.zeros((RT, CC), F32)
    zden = jnp.zeros((RT, DW), F32)

    idxs = {}
    for H in (8, 4):
        offs = jnp.arange(H, dtype=jnp.int32)[:, None] * NN
        srch = (src[None, :] + offs).reshape(H * NWK, NWIN, WIN)
        dsth = (dstg[None, :] + offs).reshape(H * NWK, NWIN, WIN)
        idxs[H] = (srch, dsth, dsts, z128, zden)

    h = _layer(x, idxs, Wl1, Wr1, att1, b1, 8, True)
    h = _layer(h, idxs, Wl2, Wr2, att2, b2, 8, True)
    h = _layer(h, idxs, Wl3, Wr3, att3, b3, 4, False)
    return h


# speedup vs baseline: 4.7114x; 1.0750x over previous
import functools

import jax
import jax.numpy as jnp
from jax import lax
from jax.experimental import pallas as pl
from jax.experimental.pallas import tpu as pltpu
from jax.experimental.pallas import tpu_sc as plsc

NN = 10000
CC = 128
EDGES = 160000
EE = EDGES + NN
NWK = 32
WIN = 128
NWIN = 42
W4 = 64
NW4 = 84
PT = WIN * NWIN
EP = NWK * PT
RA = 10112
DW = 128
RT = RA // 16
BN = 400
NB = NN // BN
EB = 1024
NEB = EP // EB
NEG = 0.2
F32 = jnp.float32


def _k1_body(x_ref, wl_ref, wr_ref, xl_ref, xr_ref):
    x = x_ref[...]
    xl_ref[...] = jnp.dot(x, wl_ref[...], preferred_element_type=F32)
    xr_ref[...] = jnp.dot(x, wr_ref[...], preferred_element_type=F32)


@functools.lru_cache(maxsize=None)
def _k1(H):
    return pl.pallas_call(
        _k1_body,
        grid=(H, NB),
        in_specs=[
            pl.BlockSpec((BN, CC), lambda h, nb: (nb, 0)),
            pl.BlockSpec((CC, CC), lambda h, nb: (0, h)),
            pl.BlockSpec((CC, CC), lambda h, nb: (0, h)),
        ],
        out_specs=[
            pl.BlockSpec((BN, CC), lambda h, nb: (h * NB + nb, 0)),
            pl.BlockSpec((BN, CC), lambda h, nb: (h * NB + nb, 0)),
        ],
        out_shape=[
            jax.ShapeDtypeStruct((H * NN, CC), F32),
            jax.ShapeDtypeStruct((H * NN, CC), F32),
        ],
    )


@functools.lru_cache(maxsize=None)
def _k2(H):
    mesh = plsc.VectorSubcoreMesh(core_axis_name="c", subcore_axis_name="s")

    @functools.partial(
        pl.kernel,
        out_type=jax.ShapeDtypeStruct((H * EP, 16), F32),
        mesh=mesh,
        scratch_types=[
            pltpu.VMEM((NWIN, WIN), jnp.int32),
            pltpu.VMEM((NWIN, WIN), jnp.int32),
            pltpu.VMEM((WIN, CC), F32),
            pltpu.VMEM((WIN, CC), F32),
            pltpu.VMEM((WIN, CC), F32),
            pltpu.VMEM((WIN, CC), F32),
            pltpu.VMEM((WIN, 16), F32),
            pltpu.VMEM((WIN, 16), F32),
            pltpu.VMEM((CC,), F32),
            pltpu.SemaphoreType.DMA,
            pltpu.SemaphoreType.DMA,
            pltpu.SemaphoreType.DMA,
            pltpu.SemaphoreType.DMA,
            pltpu.SemaphoreType.DMA,
            pltpu.SemaphoreType.DMA,
        ],
    )
    def k2(xl_hbm, xr_hbm, srch_hbm, dsth_hbm, att_hbm, ap_hbm, sidx, didx,
           bl0, bl1, br0, br1, ap0, ap1, attv, sa0, sa1, sb0, sb1, sw0, sw1):
        cid = lax.axis_index("c")
        sid = lax.axis_index("s")
        wid = cid * 16 + sid

        def head(h, carry):
            pltpu.sync_copy(srch_hbm.at[h * NWK + wid], sidx)
            pltpu.sync_copy(dsth_hbm.at[h * NWK + wid], didx)
            pltpu.sync_copy(att_hbm.at[h], attv)
            aq = [attv[pl.ds(q * 16, 16)] for q in range(CC // 16)]
            base0 = h * EP + wid * PT
            pltpu.async_copy(xl_hbm.at[sidx.at[0]], bl0, sa0)
            pltpu.async_copy(xr_hbm.at[didx.at[0]], br0, sb0)
            pltpu.async_copy(xl_hbm.at[sidx.at[1]], bl1, sa1)
            pltpu.async_copy(xr_hbm.at[didx.at[1]], br1, sb1)

            def alpha_partial(bl, br, ap):
                for j in range(WIN):
                    acc = None
                    for q in range(CC // 16):
                        sl = pl.ds(q * 16, 16)
                        v = bl[j, sl] + br[j, sl]
                        lr = jnp.where(v >= 0, v, NEG * v)
                        t = aq[q] * lr
                        acc = t if acc is None else acc + t
                    ap[j, :] = acc

            def pair(t, carry2):
                w0 = 2 * t
                w1 = w0 + 1
                pltpu.make_async_copy(xl_hbm.at[sidx.at[w0]],
                                      bl0, sa0).wait()
                pltpu.make_async_copy(xr_hbm.at[didx.at[w0]],
                                      br0, sb0).wait()

                @pl.when(t > 0)
                def _():
                    pltpu.make_async_copy(
                        ap0, ap_hbm.at[pl.ds(base0, WIN)], sw0).wait()

                alpha_partial(bl0, br0, ap0)
                pltpu.async_copy(ap0, ap_hbm.at[pl.ds(base0 + w0 * WIN,
                                                      WIN)], sw0)

                @pl.when(w0 + 2 < NWIN)
                def _():
                    pltpu.async_copy(xl_hbm.at[sidx.at[w0 + 2]], bl0, sa0)
                    pltpu.async_copy(xr_hbm.at[didx.at[w0 + 2]], br0, sb0)

                pltpu.make_async_copy(xl_hbm.at[sidx.at[w1]],
                                      bl1, sa1).wait()
                pltpu.make_async_copy(xr_hbm.at[didx.at[w1]],
                                      br1, sb1).wait()

                @pl.when(t > 0)
                def _():
                    pltpu.make_async_copy(
                        ap1, ap_hbm.at[pl.ds(base0, WIN)], sw1).wait()

                alpha_partial(bl1, br1, ap1)
                pltpu.async_copy(ap1, ap_hbm.at[pl.ds(base0 + w1 * WIN,
                                                      WIN)], sw1)

                @pl.when(w1 + 2 < NWIN)
                def _():
                    pltpu.async_copy(xl_hbm.at[sidx.at[w1 + 2]], bl1, sa1)
                    pltpu.async_copy(xr_hbm.at[didx.at[w1 + 2]], br1, sb1)

                return carry2

            lax.fori_loop(0, NWIN // 2, pair, carry)
            pltpu.make_async_copy(ap0, ap_hbm.at[pl.ds(base0, WIN)],
                                  sw0).wait()
            pltpu.make_async_copy(ap1, ap_hbm.at[pl.ds(base0, WIN)],
                                  sw1).wait()
            return carry

        lax.fori_loop(0, H, head, 0)

    return k2


def _k3a_body(ap_ref, alpha_ref, gmax_ref):
    eb = pl.program_id(1)
    s = ap_ref[...].sum(axis=1)
    alpha_ref[...] = s

    @pl.when(eb == 0)
    def _():
        gmax_ref[...] = jnp.full((CC,), -jnp.inf, F32)

    gmax_ref[...] = jnp.maximum(gmax_ref[...], jnp.full((CC,), s.max(), F32))


@functools.lru_cache(maxsize=None)
def _k3a(H):
    return pl.pallas_call(
        _k3a_body,
        grid=(H, NEB),
        in_specs=[
            pl.BlockSpec((EB, 16), lambda h, eb: (h * NEB + eb, 0)),
        ],
        out_specs=[
            pl.BlockSpec((EB,), lambda h, eb: (h * NEB + eb,)),
            pl.BlockSpec((CC,), lambda h, eb: (h,)),
        ],
        out_shape=[
            jax.ShapeDtypeStruct((H * EP,), F32),
            jax.ShapeDtypeStruct((H * CC,), F32),
        ],
    )


def _k3b_body(alpha_ref, gmax_ref, ea_ref):
    ea_ref[...] = jnp.exp(alpha_ref[...] - gmax_ref[0])


@functools.lru_cache(maxsize=None)
def _k3b(H):
    return pl.pallas_call(
        _k3b_body,
        grid=(H, NEB),
        in_specs=[
            pl.BlockSpec((EB,), lambda h, eb: (h * NEB + eb,)),
            pl.BlockSpec((CC,), lambda h, eb: (h,)),
        ],
        out_specs=pl.BlockSpec((EB,), lambda h, eb: (h * NEB + eb,)),
        out_shape=jax.ShapeDtypeStruct((H * EP,), F32),
    )


@functools.lru_cache(maxsize=None)
def _k4(H):
    mesh = plsc.VectorSubcoreMesh(core_axis_name="c", subcore_axis_name="s")

    @functools.partial(
        pl.kernel,
        out_type=jax.ShapeDtypeStruct((2 * H * RA, CC), F32),
        mesh=mesh,
        scratch_types=[
            pltpu.VMEM((NW4, W4), jnp.int32),
            pltpu.VMEM((NW4, W4), jnp.int32),
            pltpu.VMEM((NW4, W4), F32),
            pltpu.VMEM((W4, CC), F32),
            pltpu.VMEM((W4, CC), F32),
            pltpu.VMEM_SHARED((RA, CC), F32),
            pltpu.SemaphoreType.DMA,
            pltpu.SemaphoreType.DMA,
            pltpu.SemaphoreType.DMA,
            pltpu.SemaphoreType.DMA,
        ],
    )
    def k4(xl_hbm, srch_hbm, dsts_hbm, ea_hbm, z128_hbm, num_hbm,
           sidx, didx, eav, gb0, gb1, acc, sg0, sg1, ss0, ss1):
        cid = lax.axis_index("c")
        sid = lax.axis_index("s")
        wid = cid * 16 + sid
        row0 = sid * RT

        pltpu.sync_copy(dsts_hbm.at[wid], didx)

        def scale(gb, w):
            for g in range(W4 // 16):
                evec = eav[w, pl.ds(g * 16, 16)]
                for jj in range(16):
                    j = g * 16 + jj
                    ev = jnp.full((16,), evec[jj], F32)
                    for q in range(CC // 16):
                        sl = pl.ds(q * 16, 16)
                        gb[j, sl] = gb[j, sl] * ev

        def head(h, carry):
            pltpu.sync_copy(z128_hbm, acc.at[pl.ds(row0, RT)])
            pltpu.sync_copy(srch_hbm.at[h * NWK + wid], sidx)
            pltpu.sync_copy(ea_hbm.at[h * NWK + wid], eav)
            plsc.subcore_barrier()

            pltpu.async_copy(xl_hbm.at[sidx.at[0]], gb0, sg0)
            pltpu.async_copy(xl_hbm.at[sidx.at[1]], gb1, sg1)

            def pair(t, carry2):
                w0 = 2 * t
                w1 = w0 + 1
                pltpu.make_async_copy(xl_hbm.at[sidx.at[w0]], gb0, sg0).wait()
                scale(gb0, w0)
                pltpu.async_copy(gb0, acc.at[didx.at[w0]], ss0, add=True)
                pltpu.make_async_copy(xl_hbm.at[sidx.at[w1]], gb1, sg1).wait()
                scale(gb1, w1)
                pltpu.async_copy(gb1, acc.at[didx.at[w1]], ss1, add=True)
                pltpu.make_async_copy(gb0, acc.at[didx.at[w0]], ss0).wait()

                @pl.when(w0 + 2 < NW4)
                def _():
                    pltpu.async_copy(xl_hbm.at[sidx.at[w0 + 2]], gb0, sg0)

                pltpu.make_async_copy(gb1, acc.at[didx.at[w1]], ss1).wait()

                @pl.when(w1 + 2 < NW4)
                def _():
                    pltpu.async_copy(xl_hbm.at[sidx.at[w1 + 2]], gb1, sg1)

                return carry2

            lax.fori_loop(0, NW4 // 2, pair, carry)
            plsc.subcore_barrier()
            outbase = (cid * H + h) * RA + row0
            pltpu.sync_copy(acc.at[pl.ds(row0, RT)],
                            num_hbm.at[pl.ds(outbase, RT)])
            return carry

        lax.fori_loop(0, H, head, 0)

    return k4


@functools.lru_cache(maxsize=None)
def _k4d(H):
    mesh = plsc.VectorSubcoreMesh(core_axis_name="c", subcore_axis_name="s")

    @functools.partial(
        pl.kernel,
        out_type=jax.ShapeDtypeStruct((2 * H * RA, DW), F32),
        mesh=mesh,
        scratch_types=[
            pltpu.VMEM((NW4, W4), jnp.int32),
            pltpu.VMEM((NW4, W4), F32),
            pltpu.VMEM((W4, DW), F32),
            pltpu.VMEM((W4, DW), F32),
            pltpu.VMEM_SHARED((RA, DW), F32),
            pltpu.SemaphoreType.DMA,
            pltpu.SemaphoreType.DMA,
        ],
    )
    def k4d(dsts_hbm, ea_hbm, zden_hbm, den_hbm,
            didx, eav, eb0, eb1, den, ss0, ss1):
        cid = lax.axis_index("c")
        sid = lax.axis_index("s")
        wid = cid * 16 + sid
        row0 = sid * RT

        pltpu.sync_copy(dsts_hbm.at[wid], didx)

        def fill(eb, w):
            for g in range(W4 // 16):
                evec = eav[w, pl.ds(g * 16, 16)]
                for jj in range(16):
                    ev = jnp.full((16,), evec[jj], F32)
                    for q in range(DW // 16):
                        eb[g * 16 + jj, pl.ds(q * 16, 16)] = ev

        def head(h, carry):
            pltpu.sync_copy(zden_hbm, den.at[pl.ds(row0, RT)])
            pltpu.sync_copy(ea_hbm.at[h * NWK + wid], eav)
            plsc.subcore_barrier()

            def pair(t, carry2):
                w0 = 2 * t
                w1 = w0 + 1

                @pl.when(t > 0)
                def _():
                    pltpu.make_async_copy(eb0, den.at[didx.at[w0]],
                                          ss0).wait()

                fill(eb0, w0)
                pltpu.async_copy(eb0, den.at[didx.at[w0]], ss0, add=True)

                @pl.when(t > 0)
                def _():
                    pltpu.make_async_copy(eb1, den.at[didx.at[w1]],
                                          ss1).wait()

                fill(eb1, w1)
                pltpu.async_copy(eb1, den.at[didx.at[w1]], ss1, add=True)
                return carry2

            lax.fori_loop(0, NW4 // 2, pair, carry)
            pltpu.make_async_copy(eb0, den.at[didx.at[0]], ss0).wait()
            pltpu.make_async_copy(eb1, den.at[didx.at[0]], ss1).wait()
            plsc.subcore_barrier()
            outbase = (cid * H + h) * RA + row0
            pltpu.sync_copy(den.at[pl.ds(row0, RT)],
                            den_hbm.at[pl.ds(outbase, RT)])
            return carry

        lax.fori_loop(0, H, head, 0)

    return k4d


def _k5_body(num_ref, den_ref, hp_ref, b_ref, out_ref, *, relu):
    s = num_ref[...].sum(axis=0)
    d = den_ref[...].sum(axis=0)[:, :, 0:1]
    o = (s / (d + 1e-16)).mean(axis=0)
    o = o + b_ref[...]
    if relu:
        o = jnp.maximum(o, 0.0)
    out_ref[...] = o + hp_ref[...]


@functools.lru_cache(maxsize=None)
def _k5(H, relu):
    return pl.pallas_call(
        functools.partial(_k5_body, relu=relu),
        grid=(RA // 128,),
        in_specs=[
            pl.BlockSpec((2, H, 128, CC), lambda nb: (0, 0, nb, 0)),
            pl.BlockSpec((2, H, 128, DW), lambda nb: (0, 0, nb, 0)),
            pl.BlockSpec((128, CC), lambda nb: (nb, 0)),
            pl.BlockSpec((1, CC), lambda nb: (0, 0)),
        ],
        out_specs=pl.BlockSpec((128, CC), lambda nb: (nb, 0)),
        out_shape=jax.ShapeDtypeStruct((NN, CC), F32),
    )


def _layer(h_in, idxs, Wl, Wr, att, b, H, relu):
    srch, dsth, dsts, z128, zden = idxs[H]
    xl, xr = _k1(H)(h_in, Wl, Wr)
    ap = _k2(H)(xl, xr, srch, dsth, att)
    alpha, gmax = _k3a(H)(ap)
    ea = _k3b(H)(alpha, gmax)
    ea4 = ea.reshape(H * NWK, NW4, W4)
    srch4 = srch.reshape(H * NWK, NW4, W4)
    dsts4 = dsts.reshape(NWK, NW4, W4)
    num = _k4(H)(xl, srch4, dsts4, ea4, z128)
    den = _k4d(H)(dsts4, ea4, zden)
    return _k5(H, relu)(num.reshape(2, H, RA, CC),
                        den.reshape(2, H, RA, DW),
                        h_in, b.reshape(1, CC))


def kernel(x, edge_index, Wl1, Wr1, att1, b1, Wl2, Wr2, att2, b2,
           Wl3, Wr3, att3, b3):
    pad = EP - EE
    loops = jnp.arange(NN, dtype=jnp.int32)
    zpad = jnp.zeros((pad,), jnp.int32)
    src = jnp.concatenate([edge_index[0].astype(jnp.int32), loops, zpad])
    dstg = jnp.concatenate([edge_index[1].astype(jnp.int32), loops, zpad])
    dsts = jnp.concatenate([edge_index[1].astype(jnp.int32), loops,
                            jnp.full((pad,), NN, jnp.int32)])
    dsts = dsts.reshape(NWK, NWIN, WIN)
    z128 = jnp.zeros((RT, CC), F32)
    zden = jnp.zeros((RT, DW), F32)

    idxs = {}
    for H in (8, 4):
        offs = jnp.arange(H, dtype=jnp.int32)[:, None] * NN
        srch = (src[None, :] + offs).reshape(H * NWK, NWIN, WIN)
        dsth = (dstg[None, :] + offs).reshape(H * NWK, NWIN, WIN)
        idxs[H] = (srch, dsth, dsts, z128, zden)

    h = _layer(x, idxs, Wl1, Wr1, att1, b1, 8, True)
    h = _layer(h, idxs, Wl2, Wr2, att2, b2, 8, True)
    h = _layer(h, idxs, Wl3, Wr3, att3, b3, 4, False)
    return h
